# baseline (speedup 1.0000x reference)
import jax
import jax.numpy as jnp
from jax.experimental import pallas as pl
from jax.experimental.pallas import tpu as pltpu

N = 16384
M_DIM = 512
CTRL = 1024
OUT_F = 3 * M_DIM + 6
EPS = 1e-16

W_SPLIT = 520

C0 = [(0, 4096), (4096, 4096), (8192, 4096), (12288, 2048), (14336, 2048)]
NS0 = len(C0)
C1 = [(0, 1024), (1024, 1024), (2048, 2048), (4096, 4096),
      (8192, 4096), (12288, 2048), (14336, 1024), (15360, 1024)]


def _sim_chunk(o_sc, mem_vmem, sim_sc, off, sz):
    mem = mem_vmem[off:off + sz, :]
    k = o_sc[:, :M_DIM]
    kn = jnp.sqrt(jnp.sum(k * k, axis=1, keepdims=True))
    dot = jax.lax.dot_general(
        k, mem,
        dimension_numbers=(((1,), (1,)), ((), ())),
        preferred_element_type=jnp.float32,
    )
    ones = jnp.ones((1, M_DIM), dtype=jnp.float32)
    rn2 = jax.lax.dot_general(
        ones, mem * mem,
        dimension_numbers=(((1,), (1,)), ((), ())),
        preferred_element_type=jnp.float32,
    )
    sim_sc[:, off:off + sz] = dot / (kn * jnp.sqrt(rn2) + EPS)


def _wh_kernel(emb_ref, w_hbm, b_ref, wprev_ref, mem_hbm,
               w_out, memout_ref,
               o_sc, sim_sc, mem_vmem, w_vmem, w1_sem, w2_sem, mem_sems,
               out_sems):
    s = pl.program_id(0)

    @pl.when(s == 0)
    def _prologue():
        pltpu.make_async_copy(w_hbm.at[pl.ds(0, W_SPLIT), :],
                              w_vmem.at[pl.ds(0, W_SPLIT), :], w1_sem).start()
        for j, (off, sz) in enumerate(C0):
            pltpu.make_async_copy(mem_hbm.at[pl.ds(off, sz), :],
                                  mem_vmem.at[pl.ds(off, sz), :],
                                  mem_sems.at[j]).start()
        pltpu.make_async_copy(w_hbm.at[pl.ds(W_SPLIT, OUT_F - W_SPLIT), :],
                              w_vmem.at[pl.ds(W_SPLIT, OUT_F - W_SPLIT), :],
                              w2_sem).start()
        pltpu.make_async_copy(w_hbm.at[pl.ds(0, W_SPLIT), :],
                              w_vmem.at[pl.ds(0, W_SPLIT), :], w1_sem).wait()
        o_sc[:, :W_SPLIT] = jax.lax.dot_general(
            emb_ref[...], w_vmem[:W_SPLIT, :],
            dimension_numbers=(((1,), (1,)), ((), ())),
            preferred_element_type=jnp.float32,
        ) + b_ref[:, :W_SPLIT]

    for j, (off, sz) in enumerate(C0):
        @pl.when(s == j)
        def _sim_phase(j=j, off=off, sz=sz):
            pltpu.make_async_copy(mem_hbm.at[pl.ds(off, sz), :],
                                  mem_vmem.at[pl.ds(off, sz), :],
                                  mem_sems.at[j]).wait()
            _sim_chunk(o_sc, mem_vmem, sim_sc, off, sz)

    @pl.when(s == NS0)
    def _final():
        o = o_sc[...]
        beta = jax.nn.softplus(o[:, M_DIM:M_DIM + 1])
        g = jax.nn.sigmoid(o[:, M_DIM + 1:M_DIM + 2])
        sv = jax.nn.softmax(o[:, M_DIM + 2:M_DIM + 5], axis=1)
        gamma = 1.0 + jax.nn.softplus(o[:, M_DIM + 5:M_DIM + 6])

        z = beta * sim_sc[...]
        m = jnp.max(z, axis=1, keepdims=True)
        ez = jnp.exp(z - m)
        wc = ez / jnp.sum(ez, axis=1, keepdims=True)

        wg = g * wc + (1.0 - g) * wprev_ref[...]

        roll_p = jnp.concatenate([wg[:, -1:], wg[:, :-1]], axis=1)
        roll_m = jnp.concatenate([wg[:, 1:], wg[:, :1]], axis=1)
        ws = sv[:, 0:1] * roll_p + sv[:, 1:2] * wg + sv[:, 2:3] * roll_m

        wp = jnp.exp(gamma * jnp.log(ws + EPS))
        w_out[...] = wp / jnp.sum(wp, axis=1, keepdims=True)

        pltpu.make_async_copy(w_hbm.at[pl.ds(W_SPLIT, OUT_F - W_SPLIT), :],
                              w_vmem.at[pl.ds(W_SPLIT, OUT_F - W_SPLIT), :],
                              w2_sem).wait()
        o_sc[:, W_SPLIT:] = jax.lax.dot_general(
            emb_ref[...], w_vmem[W_SPLIT:, :],
            dimension_numbers=(((1,), (1,)), ((), ())),
            preferred_element_type=jnp.float32,
        ) + b_ref[:, W_SPLIT:]

        e = o_sc[:, M_DIM + 6:2 * M_DIM + 6].astype(jnp.bfloat16)
        a = o_sc[:, 2 * M_DIM + 6:].astype(jnp.bfloat16)
        for j, (off, sz) in enumerate(C1):
            wb = w_out[:, off:off + sz].astype(jnp.bfloat16)
            ers = jax.lax.dot_general(
                wb, e,
                dimension_numbers=(((0,), (0,)), ((), ())),
                preferred_element_type=jnp.float32,
            )
            ads = jax.lax.dot_general(
                wb, a,
                dimension_numbers=(((0,), (0,)), ((), ())),
                preferred_element_type=jnp.float32,
            )
            mem = mem_vmem[off:off + sz, :]
            mem_vmem[off:off + sz, :] = mem - mem * ers + ads
            pltpu.make_async_copy(mem_vmem.at[pl.ds(off, sz), :],
                                  memout_ref.at[pl.ds(off, sz), :],
                                  out_sems.at[j]).start()
        for j, (off, sz) in enumerate(C1):
            pltpu.make_async_copy(mem_vmem.at[pl.ds(off, sz), :],
                                  memout_ref.at[pl.ds(off, sz), :],
                                  out_sems.at[j]).wait()


def kernel(embeddings, w_prev, memory, W, b):
    b2d = b.reshape(1, OUT_F)

    w, new_memory = pl.pallas_call(
        _wh_kernel,
        grid=(NS0 + 1,),
        in_specs=[
            pl.BlockSpec((1, CTRL), lambda s: (0, 0)),
            pl.BlockSpec(memory_space=pl.ANY),
            pl.BlockSpec((1, OUT_F), lambda s: (0, 0)),
            pl.BlockSpec((1, N), lambda s: (0, 0)),
            pl.BlockSpec(memory_space=pl.ANY),
        ],
        out_specs=(
            pl.BlockSpec((1, N), lambda s: (0, 0)),
            pl.BlockSpec(memory_space=pl.ANY),
        ),
        out_shape=(
            jax.ShapeDtypeStruct((1, N), jnp.float32),
            jax.ShapeDtypeStruct((N, M_DIM), jnp.float32),
        ),
        scratch_shapes=[
            pltpu.VMEM((1, OUT_F), jnp.float32),
            pltpu.VMEM((1, N), jnp.float32),
            pltpu.VMEM((N, M_DIM), jnp.float32),
            pltpu.VMEM((OUT_F, CTRL), jnp.float32),
            pltpu.SemaphoreType.DMA,
            pltpu.SemaphoreType.DMA,
            pltpu.SemaphoreType.DMA((NS0,)),
            pltpu.SemaphoreType.DMA((len(C1),)),
        ],
        compiler_params=pltpu.CompilerParams(
            dimension_semantics=("arbitrary",),
            vmem_limit_bytes=56 * 1024 * 1024,
        ),
        name="wh_fused",
    )(embeddings, W, b2d, w_prev, memory)

    return w, new_memory
